# baseline (speedup 1.0000x reference)
import functools

import jax
import jax.numpy as jnp
from jax import lax
from jax.experimental import pallas as pl
from jax.experimental.pallas import tpu as pltpu

B, N, HID = 16, 2048, 256
ROWS = B * N
BLK = 2048
NBLK = ROWS // BLK
NEG = -1e30


def _copy(emb_hbm, esave_ref, sems, blk):
    return pltpu.make_async_copy(
        emb_hbm.at[pl.ds(blk * BLK, BLK), :],
        esave_ref.at[pl.ds(blk * BLK, BLK), :],
        sems.at[blk % 2],
    )


def _body(emb_hbm, plen_ref, w1t_ref, b1_ref, w2c_ref,
          out_ref, m_ref, mblk_ref, zblk_ref, pv_ref, esave_ref,
          pooled_ref, sems):
    j = pl.program_id(0)
    i = pl.program_id(1)

    @pl.when(j == 0)
    def _accumulate():
        @pl.when(i == 0)
        def _init():
            m_ref[0, 0] = NEG
            _copy(emb_hbm, esave_ref, sems, 0).start()

        @pl.when(i + 1 < NBLK)
        def _prefetch():
            _copy(emb_hbm, esave_ref, sems, i + 1).start()

        _copy(emb_hbm, esave_ref, sems, i).wait()
        e = esave_ref[pl.ds(i * BLK, BLK), :]
        plen = plen_ref[...]
        h = jnp.tanh(jnp.dot(e, w1t_ref[...],
                             preferred_element_type=jnp.float32) + b1_ref[...])
        s = jnp.dot(h, w2c_ref[...],
                    preferred_element_type=jnp.float32)
        med = (plen >= 3) & (plen < 5)
        sm = jnp.where(med, s, NEG)
        bm = jnp.max(sm)
        m_old = m_ref[0, 0]
        m_new = jnp.maximum(m_old, bm)
        m_ref[0, 0] = m_new
        mblk_ref[i, 0] = m_new
        p = jnp.where(med, jnp.exp(sm - m_new), 0.0)
        zblk_ref[i, 0] = jnp.sum(p)
        pv_ref[pl.ds(i, 1), :] = lax.dot_general(
            p, e, (((0,), (0,)), ((), ())),
            preferred_element_type=jnp.float32)

        @pl.when(i == NBLK - 1)
        def _fin():
            m_fin = m_ref[0, 0]
            z = 0.0
            v = jnp.zeros((1, HID), jnp.float32)
            for k in range(NBLK):
                c = jnp.exp(mblk_ref[k, 0] - m_fin)
                z = z + zblk_ref[k, 0] * c
                v = v + pv_ref[pl.ds(k, 1), :] * c
            pooled_ref[...] = v / z

    @pl.when(j == 1)
    def _emit():
        e = esave_ref[pl.ds(i * BLK, BLK), :]
        plen = plen_ref[...]
        med = (plen >= 3) & (plen < 5)
        short = plen < 3
        out_ref[...] = jnp.where(short, e,
                                 jnp.where(med, pooled_ref[...], 0.0))


def kernel(embeddings, path_lengths, W1, b1, W2, b2):
    del b2
    emb2 = embeddings.reshape(ROWS, HID)
    plen2 = path_lengths.reshape(ROWS, 1)
    w1t = W1.T
    b1r = b1.reshape(1, HID // 2)
    w2c = W2.reshape(HID // 2, 1)

    out = pl.pallas_call(
        _body,
        grid=(2, NBLK),
        in_specs=[
            pl.BlockSpec(memory_space=pl.ANY),
            pl.BlockSpec((BLK, 1), lambda j, i: (i, 0)),
            pl.BlockSpec((HID, HID // 2), lambda j, i: (0, 0)),
            pl.BlockSpec((1, HID // 2), lambda j, i: (0, 0)),
            pl.BlockSpec((HID // 2, 1), lambda j, i: (0, 0)),
        ],
        out_specs=pl.BlockSpec((BLK, HID),
                               lambda j, i: (jnp.where(j == 0, 0, i), 0)),
        out_shape=jax.ShapeDtypeStruct((ROWS, HID), jnp.float32),
        scratch_shapes=[
            pltpu.SMEM((1, 1), jnp.float32),
            pltpu.SMEM((NBLK, 1), jnp.float32),
            pltpu.SMEM((NBLK, 1), jnp.float32),
            pltpu.VMEM((NBLK, HID), jnp.float32),
            pltpu.VMEM((ROWS, HID), jnp.float32),
            pltpu.VMEM((1, HID), jnp.float32),
            pltpu.SemaphoreType.DMA((2,)),
        ],
        compiler_params=pltpu.CompilerParams(
            dimension_semantics=("arbitrary", "arbitrary"),
        ),
    )(emb2, plen2, w1t, b1r, w2c)

    return out.reshape(B, N, HID)


# speedup vs baseline: 1.0186x; 1.0186x over previous
import functools

import jax
import jax.numpy as jnp
from jax import lax
from jax.experimental import pallas as pl
from jax.experimental.pallas import tpu as pltpu

B, N, HID = 16, 2048, 256
ROWS = B * N
BLK = 2048
NBLK = ROWS // BLK
NEG = -1e30


def _copy_in(emb_hbm, esave_ref, sems, blk):
    return pltpu.make_async_copy(
        emb_hbm.at[pl.ds(blk * BLK, BLK), :],
        esave_ref.at[pl.ds(blk * BLK, BLK), :],
        sems.at[blk % 2],
    )


def _copy_out(esave_ref, out_hbm, sems, blk):
    return pltpu.make_async_copy(
        esave_ref.at[pl.ds(blk * BLK, BLK), :],
        out_hbm.at[pl.ds(blk * BLK, BLK), :],
        sems.at[blk % 2],
    )


def _body(emb_hbm, plen_ref, w1t_ref, b1_ref, w2c_ref,
          out_hbm, mblk_ref, zblk_ref, pv_ref, esave_ref,
          pooled_ref, sems_in, sems_out):
    j = pl.program_id(0)
    i = pl.program_id(1)

    @pl.when(j == 0)
    def _accumulate():
        @pl.when(i == 0)
        def _init():
            _copy_in(emb_hbm, esave_ref, sems_in, 0).start()

        @pl.when(i + 1 < NBLK)
        def _prefetch():
            _copy_in(emb_hbm, esave_ref, sems_in, i + 1).start()

        _copy_in(emb_hbm, esave_ref, sems_in, i).wait()
        e = esave_ref[pl.ds(i * BLK, BLK), :]
        plen = plen_ref[...]
        h = jnp.tanh(jnp.dot(e, w1t_ref[...],
                             preferred_element_type=jnp.float32) + b1_ref[...])
        s = jnp.dot(h, w2c_ref[...],
                    preferred_element_type=jnp.float32)
        med = (plen >= 3) & (plen < 5)
        sm = jnp.where(med, s, NEG)
        bm = jnp.max(sm)
        mblk_ref[i, 0] = bm
        p = jnp.exp(sm - bm)
        zblk_ref[i, 0] = jnp.sum(p)
        pv_ref[pl.ds(i, 1), :] = lax.dot_general(
            p, e, (((0,), (0,)), ((), ())),
            preferred_element_type=jnp.float32)

        @pl.when(i == NBLK - 1)
        def _fin():
            m_fin = mblk_ref[0, 0]
            for k in range(1, NBLK):
                m_fin = jnp.maximum(m_fin, mblk_ref[k, 0])
            z = 0.0
            v = jnp.zeros((1, HID), jnp.float32)
            for k in range(NBLK):
                c = jnp.exp(mblk_ref[k, 0] - m_fin)
                z = z + zblk_ref[k, 0] * c
                v = v + pv_ref[pl.ds(k, 1), :] * c
            pooled_ref[...] = v / z

    @pl.when(j == 1)
    def _emit():
        e = esave_ref[pl.ds(i * BLK, BLK), :]
        plen = plen_ref[...]
        med = (plen >= 3) & (plen < 5)
        short = plen < 3
        esave_ref[pl.ds(i * BLK, BLK), :] = jnp.where(
            short, e, jnp.where(med, pooled_ref[...], 0.0))
        _copy_out(esave_ref, out_hbm, sems_out, i).start()

        @pl.when(i == NBLK - 1)
        def _drain():
            for k in range(NBLK):
                _copy_out(esave_ref, out_hbm, sems_out, k).wait()


def kernel(embeddings, path_lengths, W1, b1, W2, b2):
    del b2
    emb2 = embeddings.reshape(ROWS, HID)
    plen2 = path_lengths.reshape(ROWS, 1)
    w1t = W1.T
    b1r = b1.reshape(1, HID // 2)
    w2c = W2.reshape(HID // 2, 1)

    out = pl.pallas_call(
        _body,
        grid=(2, NBLK),
        in_specs=[
            pl.BlockSpec(memory_space=pl.ANY),
            pl.BlockSpec((BLK, 1), lambda j, i: (i, 0)),
            pl.BlockSpec((HID, HID // 2), lambda j, i: (0, 0)),
            pl.BlockSpec((1, HID // 2), lambda j, i: (0, 0)),
            pl.BlockSpec((HID // 2, 1), lambda j, i: (0, 0)),
        ],
        out_specs=pl.BlockSpec(memory_space=pl.ANY),
        out_shape=jax.ShapeDtypeStruct((ROWS, HID), jnp.float32),
        scratch_shapes=[
            pltpu.SMEM((NBLK, 1), jnp.float32),
            pltpu.SMEM((NBLK, 1), jnp.float32),
            pltpu.VMEM((NBLK, HID), jnp.float32),
            pltpu.VMEM((ROWS, HID), jnp.float32),
            pltpu.VMEM((1, HID), jnp.float32),
            pltpu.SemaphoreType.DMA((2,)),
            pltpu.SemaphoreType.DMA((2,)),
        ],
        compiler_params=pltpu.CompilerParams(
            dimension_semantics=("arbitrary", "arbitrary"),
        ),
    )(emb2, plen2, w1t, b1r, w2c)

    return out.reshape(B, N, HID)


# speedup vs baseline: 1.1715x; 1.1501x over previous
import functools

import jax
import jax.numpy as jnp
from jax import lax
from jax.experimental import pallas as pl
from jax.experimental.pallas import tpu as pltpu

B, N, HID = 16, 2048, 256
ROWS = B * N
BLK = 4096
NBLK = ROWS // BLK
NEG = -1e30


def _copy_in(emb_hbm, esave_ref, sems, blk):
    return pltpu.make_async_copy(
        emb_hbm.at[pl.ds(blk * BLK, BLK), :],
        esave_ref.at[pl.ds(blk * BLK, BLK), :],
        sems.at[blk % 2],
    )


def _copy_out(esave_ref, out_hbm, sems, blk):
    return pltpu.make_async_copy(
        esave_ref.at[pl.ds(blk * BLK, BLK), :],
        out_hbm.at[pl.ds(blk * BLK, BLK), :],
        sems.at[blk % 2],
    )


def _body(emb_hbm, plen_ref, w1t_ref, b1_ref, w2c_ref,
          out_hbm, mblk_ref, zblk_ref, pv_ref, esave_ref,
          pooled_ref, sems_in, sems_out):
    j = pl.program_id(0)
    i = pl.program_id(1)

    @pl.when(j == 0)
    def _accumulate():
        @pl.when(i == 0)
        def _init():
            _copy_in(emb_hbm, esave_ref, sems_in, 0).start()

        @pl.when(i + 1 < NBLK)
        def _prefetch():
            _copy_in(emb_hbm, esave_ref, sems_in, i + 1).start()

        _copy_in(emb_hbm, esave_ref, sems_in, i).wait()
        e = esave_ref[pl.ds(i * BLK, BLK), :]
        plen = plen_ref[...]
        h = jnp.tanh(jnp.dot(e, w1t_ref[...],
                             preferred_element_type=jnp.float32) + b1_ref[...])
        s = jnp.dot(h, w2c_ref[...],
                    preferred_element_type=jnp.float32)
        med = (plen >= 3) & (plen < 5)
        sm = jnp.where(med, s, NEG)
        bm = jnp.max(sm)
        mblk_ref[i, 0] = bm
        p = jnp.exp(sm - bm)
        zblk_ref[i, 0] = jnp.sum(p)
        pv_ref[pl.ds(i, 1), :] = lax.dot_general(
            p, e, (((0,), (0,)), ((), ())),
            preferred_element_type=jnp.float32)

        @pl.when(i == NBLK - 1)
        def _fin():
            m_fin = mblk_ref[0, 0]
            for k in range(1, NBLK):
                m_fin = jnp.maximum(m_fin, mblk_ref[k, 0])
            z = 0.0
            v = jnp.zeros((1, HID), jnp.float32)
            for k in range(NBLK):
                c = jnp.exp(mblk_ref[k, 0] - m_fin)
                z = z + zblk_ref[k, 0] * c
                v = v + pv_ref[pl.ds(k, 1), :] * c
            pooled_ref[...] = v / z

    @pl.when(j == 1)
    def _emit():
        e = esave_ref[pl.ds(i * BLK, BLK), :]
        plen = plen_ref[...]
        med = (plen >= 3) & (plen < 5)
        short = plen < 3
        esave_ref[pl.ds(i * BLK, BLK), :] = jnp.where(
            short, e, jnp.where(med, pooled_ref[...], 0.0))
        _copy_out(esave_ref, out_hbm, sems_out, i).start()

        @pl.when(i == NBLK - 1)
        def _drain():
            for k in range(NBLK):
                _copy_out(esave_ref, out_hbm, sems_out, k).wait()


def kernel(embeddings, path_lengths, W1, b1, W2, b2):
    del b2
    emb2 = embeddings.reshape(ROWS, HID)
    plen2 = path_lengths.reshape(ROWS, 1)
    w1t = W1.T
    b1r = b1.reshape(1, HID // 2)
    w2c = W2.reshape(HID // 2, 1)

    out = pl.pallas_call(
        _body,
        grid=(2, NBLK),
        in_specs=[
            pl.BlockSpec(memory_space=pl.ANY),
            pl.BlockSpec((BLK, 1), lambda j, i: (i, 0)),
            pl.BlockSpec((HID, HID // 2), lambda j, i: (0, 0)),
            pl.BlockSpec((1, HID // 2), lambda j, i: (0, 0)),
            pl.BlockSpec((HID // 2, 1), lambda j, i: (0, 0)),
        ],
        out_specs=pl.BlockSpec(memory_space=pl.ANY),
        out_shape=jax.ShapeDtypeStruct((ROWS, HID), jnp.float32),
        scratch_shapes=[
            pltpu.SMEM((NBLK, 1), jnp.float32),
            pltpu.SMEM((NBLK, 1), jnp.float32),
            pltpu.VMEM((NBLK, HID), jnp.float32),
            pltpu.VMEM((ROWS, HID), jnp.float32),
            pltpu.VMEM((1, HID), jnp.float32),
            pltpu.SemaphoreType.DMA((2,)),
            pltpu.SemaphoreType.DMA((2,)),
        ],
        compiler_params=pltpu.CompilerParams(
            dimension_semantics=("arbitrary", "arbitrary"),
        ),
    )(emb2, plen2, w1t, b1r, w2c)

    return out.reshape(B, N, HID)


# speedup vs baseline: 1.2711x; 1.0850x over previous
import functools

import jax
import jax.numpy as jnp
from jax import lax
from jax.experimental import pallas as pl
from jax.experimental.pallas import tpu as pltpu

B, N, HID = 16, 2048, 256
ROWS = B * N
BLK = 8192
NBLK = ROWS // BLK
NEG = -1e30


def _copy_in(emb_hbm, esave_ref, sems, blk):
    return pltpu.make_async_copy(
        emb_hbm.at[pl.ds(blk * BLK, BLK), :],
        esave_ref.at[pl.ds(blk * BLK, BLK), :],
        sems.at[blk % 2],
    )


def _copy_out(esave_ref, out_hbm, sems, blk):
    return pltpu.make_async_copy(
        esave_ref.at[pl.ds(blk * BLK, BLK), :],
        out_hbm.at[pl.ds(blk * BLK, BLK), :],
        sems.at[blk % 2],
    )


def _body(emb_hbm, plen_ref, w1t_ref, b1_ref, w2c_ref,
          out_hbm, mblk_ref, zblk_ref, pv_ref, esave_ref,
          pooled_ref, sems_in, sems_out):
    j = pl.program_id(0)
    i = pl.program_id(1)

    @pl.when(j == 0)
    def _accumulate():
        @pl.when(i == 0)
        def _init():
            _copy_in(emb_hbm, esave_ref, sems_in, 0).start()

        @pl.when(i + 1 < NBLK)
        def _prefetch():
            _copy_in(emb_hbm, esave_ref, sems_in, i + 1).start()

        _copy_in(emb_hbm, esave_ref, sems_in, i).wait()
        e = esave_ref[pl.ds(i * BLK, BLK), :]
        plen = plen_ref[...]
        h = jnp.tanh(jnp.dot(e, w1t_ref[...],
                             preferred_element_type=jnp.float32) + b1_ref[...])
        s = jnp.dot(h, w2c_ref[...],
                    preferred_element_type=jnp.float32)
        med = (plen >= 3) & (plen < 5)
        sm = jnp.where(med, s, NEG)
        bm = jnp.max(sm)
        mblk_ref[i, 0] = bm
        p = jnp.exp(sm - bm)
        zblk_ref[i, 0] = jnp.sum(p)
        pv_ref[pl.ds(i, 1), :] = lax.dot_general(
            p, e, (((0,), (0,)), ((), ())),
            preferred_element_type=jnp.float32)

        @pl.when(i == NBLK - 1)
        def _fin():
            m_fin = mblk_ref[0, 0]
            for k in range(1, NBLK):
                m_fin = jnp.maximum(m_fin, mblk_ref[k, 0])
            z = 0.0
            v = jnp.zeros((1, HID), jnp.float32)
            for k in range(NBLK):
                c = jnp.exp(mblk_ref[k, 0] - m_fin)
                z = z + zblk_ref[k, 0] * c
                v = v + pv_ref[pl.ds(k, 1), :] * c
            pooled_ref[...] = v / z

    @pl.when(j == 1)
    def _emit():
        e = esave_ref[pl.ds(i * BLK, BLK), :]
        plen = plen_ref[...]
        med = (plen >= 3) & (plen < 5)
        short = plen < 3
        esave_ref[pl.ds(i * BLK, BLK), :] = jnp.where(
            short, e, jnp.where(med, pooled_ref[...], 0.0))
        _copy_out(esave_ref, out_hbm, sems_out, i).start()

        @pl.when(i == NBLK - 1)
        def _drain():
            for k in range(NBLK):
                _copy_out(esave_ref, out_hbm, sems_out, k).wait()


def kernel(embeddings, path_lengths, W1, b1, W2, b2):
    del b2
    emb2 = embeddings.reshape(ROWS, HID)
    plen2 = path_lengths.reshape(ROWS, 1)
    w1t = W1.T
    b1r = b1.reshape(1, HID // 2)
    w2c = W2.reshape(HID // 2, 1)

    out = pl.pallas_call(
        _body,
        grid=(2, NBLK),
        in_specs=[
            pl.BlockSpec(memory_space=pl.ANY),
            pl.BlockSpec((BLK, 1), lambda j, i: (i, 0)),
            pl.BlockSpec((HID, HID // 2), lambda j, i: (0, 0)),
            pl.BlockSpec((1, HID // 2), lambda j, i: (0, 0)),
            pl.BlockSpec((HID // 2, 1), lambda j, i: (0, 0)),
        ],
        out_specs=pl.BlockSpec(memory_space=pl.ANY),
        out_shape=jax.ShapeDtypeStruct((ROWS, HID), jnp.float32),
        scratch_shapes=[
            pltpu.SMEM((NBLK, 1), jnp.float32),
            pltpu.SMEM((NBLK, 1), jnp.float32),
            pltpu.VMEM((NBLK, HID), jnp.float32),
            pltpu.VMEM((ROWS, HID), jnp.float32),
            pltpu.VMEM((1, HID), jnp.float32),
            pltpu.SemaphoreType.DMA((2,)),
            pltpu.SemaphoreType.DMA((2,)),
        ],
        compiler_params=pltpu.CompilerParams(
            dimension_semantics=("arbitrary", "arbitrary"),
        ),
    )(emb2, plen2, w1t, b1r, w2c)

    return out.reshape(B, N, HID)


# speedup vs baseline: 1.2918x; 1.0163x over previous
import functools

import jax
import jax.numpy as jnp
from jax import lax
from jax.experimental import pallas as pl
from jax.experimental.pallas import tpu as pltpu

B, N, HID = 16, 2048, 256
ROWS = B * N
BLK = 8192
NBLK = ROWS // BLK
NEG = -1e30


def _copy_in(emb_hbm, esave_ref, sems, blk):
    return pltpu.make_async_copy(
        emb_hbm.at[pl.ds(blk * BLK, BLK), :],
        esave_ref.at[pl.ds(blk * BLK, BLK), :],
        sems.at[blk],
    )


def _copy_out(esave_ref, out_hbm, sems, blk):
    return pltpu.make_async_copy(
        esave_ref.at[pl.ds(blk * BLK, BLK), :],
        out_hbm.at[pl.ds(blk * BLK, BLK), :],
        sems.at[blk],
    )


def _body(emb_hbm, plen_ref, w1t_ref, b1_ref, w2c_ref,
          out_hbm, mblk_ref, zblk_ref, pv_ref, esave_ref,
          pooled_ref, sems_in, sems_out):
    j = pl.program_id(0)
    i = pl.program_id(1)

    @pl.when(j == 0)
    def _accumulate():
        @pl.when(i == 0)
        def _init():
            for k in range(NBLK):
                _copy_in(emb_hbm, esave_ref, sems_in, k).start()

        _copy_in(emb_hbm, esave_ref, sems_in, i).wait()
        e = esave_ref[pl.ds(i * BLK, BLK), :]
        eb = e.astype(jnp.bfloat16)
        plen = plen_ref[...]
        h = jnp.tanh(jnp.dot(eb, w1t_ref[...],
                             preferred_element_type=jnp.float32) + b1_ref[...])
        s = jnp.dot(h, w2c_ref[...],
                    preferred_element_type=jnp.float32)
        med = (plen >= 3) & (plen < 5)
        sm = jnp.where(med, s, NEG)
        bm = jnp.max(sm)
        mblk_ref[i, 0] = bm
        p = jnp.exp(sm - bm)
        zblk_ref[i, 0] = jnp.sum(p)
        pv_ref[pl.ds(i, 1), :] = lax.dot_general(
            p.astype(jnp.bfloat16), eb, (((0,), (0,)), ((), ())),
            preferred_element_type=jnp.float32)

        @pl.when(i == NBLK - 1)
        def _fin():
            m_fin = mblk_ref[0, 0]
            for k in range(1, NBLK):
                m_fin = jnp.maximum(m_fin, mblk_ref[k, 0])
            z = 0.0
            v = jnp.zeros((1, HID), jnp.float32)
            for k in range(NBLK):
                c = jnp.exp(mblk_ref[k, 0] - m_fin)
                z = z + zblk_ref[k, 0] * c
                v = v + pv_ref[pl.ds(k, 1), :] * c
            pooled_ref[...] = v * jnp.where(z > 0, 1.0 / z, 0.0)

    @pl.when(j == 1)
    def _emit():
        e = esave_ref[pl.ds(i * BLK, BLK), :]
        plen = plen_ref[...]
        short_f = (plen < 3).astype(jnp.float32)
        med_f = ((plen >= 3) & (plen < 5)).astype(jnp.float32)
        esave_ref[pl.ds(i * BLK, BLK), :] = (
            e * short_f + med_f * pooled_ref[...])
        _copy_out(esave_ref, out_hbm, sems_out, i).start()

        @pl.when(i == NBLK - 1)
        def _drain():
            for k in range(NBLK):
                _copy_out(esave_ref, out_hbm, sems_out, k).wait()


def kernel(embeddings, path_lengths, W1, b1, W2, b2):
    del b2
    emb2 = embeddings.reshape(ROWS, HID)
    plen2 = path_lengths.reshape(ROWS, 1)
    w1t = W1.T.astype(jnp.bfloat16)
    b1r = b1.reshape(1, HID // 2)
    w2c = W2.reshape(HID // 2, 1)

    out = pl.pallas_call(
        _body,
        grid=(2, NBLK),
        in_specs=[
            pl.BlockSpec(memory_space=pl.ANY),
            pl.BlockSpec((BLK, 1), lambda j, i: (i, 0)),
            pl.BlockSpec((HID, HID // 2), lambda j, i: (0, 0)),
            pl.BlockSpec((1, HID // 2), lambda j, i: (0, 0)),
            pl.BlockSpec((HID // 2, 1), lambda j, i: (0, 0)),
        ],
        out_specs=pl.BlockSpec(memory_space=pl.ANY),
        out_shape=jax.ShapeDtypeStruct((ROWS, HID), jnp.float32),
        scratch_shapes=[
            pltpu.SMEM((NBLK, 1), jnp.float32),
            pltpu.SMEM((NBLK, 1), jnp.float32),
            pltpu.VMEM((NBLK, HID), jnp.float32),
            pltpu.VMEM((ROWS, HID), jnp.float32),
            pltpu.VMEM((1, HID), jnp.float32),
            pltpu.SemaphoreType.DMA((NBLK,)),
            pltpu.SemaphoreType.DMA((NBLK,)),
        ],
        compiler_params=pltpu.CompilerParams(
            dimension_semantics=("arbitrary", "arbitrary"),
        ),
    )(emb2, plen2, w1t, b1r, w2c)

    return out.reshape(B, N, HID)
